# Initial kernel scaffold; baseline (speedup 1.0000x reference)
import functools

import jax
import jax.numpy as jnp
from jax import lax
from jax.experimental import pallas as pl
from jax.experimental.pallas import tpu as pltpu
from jax.experimental.pallas import tpu_sc as plsc

N = 10000
E = 320000
G = 64
D1 = 336
D2 = 168
CHUNK = 128
NCHUNKS = E // CHUNK
NC = 2
NS = 16
ROWS_PER_TILE = N // NS
ZROWS = 125
RBLK = 1000
NBLK = N // RBLK

_mesh = functools.partial(
    plsc.VectorSubcoreMesh, core_axis_name="c", subcore_axis_name="s",
    num_cores=NC, num_subcores=NS)


def _my_chunk_range(sid, per_sc, sc_chunk0):
  per, rem = divmod(per_sc, NS)
  n_my = per + jnp.where(sid < rem, 1, 0)
  base = sc_chunk0 + sid * per + jnp.minimum(sid, rem)
  return base, n_my


def _deg_body(dst_hbm, ones_hbm, zeros_hbm, out_hbm,
              idx_v, ones_v, zstage_v, acc_s):
  c = lax.axis_index("c")
  s = lax.axis_index("s")
  wid = c * NS + s
  pltpu.sync_copy(ones_hbm, ones_v)
  pltpu.sync_copy(zeros_hbm, zstage_v)
  r0 = s * ROWS_PER_TILE
  for k in range(ROWS_PER_TILE // ZROWS):
    pltpu.sync_copy(zstage_v, acc_s.at[pl.ds(r0 + k * ZROWS, ZROWS)])
  plsc.subcore_barrier()

  base, n_my = _my_chunk_range(wid, NCHUNKS, 0)

  def chunk_body(i, carry):
    cb = base + i
    pltpu.sync_copy(dst_hbm.at[cb], idx_v)
    pltpu.sync_copy(ones_v, acc_s.at[idx_v], add=True)
    return carry

  lax.fori_loop(0, n_my, chunk_body, 0)
  plsc.subcore_barrier()

  for k in range(ROWS_PER_TILE // ZROWS):
    rr = r0 + k * ZROWS
    pltpu.sync_copy(acc_s.at[pl.ds(rr, ZROWS)], zstage_v)
    pltpu.sync_copy(zstage_v, out_hbm.at[c, pl.ds(rr, ZROWS)])


def _deg_call(dst2d, ones_arr, zeros_arr):
  return pl.kernel(
      _deg_body,
      out_type=jax.ShapeDtypeStruct((NC, N, 16), jnp.float32),
      mesh=_mesh(),
      scratch_types=[
          pltpu.VMEM((CHUNK,), jnp.int32),
          pltpu.VMEM((CHUNK, 16), jnp.float32),
          pltpu.VMEM((ZROWS, 16), jnp.float32),
          pltpu.VMEM_SHARED((N, 16), jnp.float32),
      ],
      name="sc_deg_hist",
  )(dst2d, ones_arr, zeros_arr)


def _mp_body(col_split, src_hbm, dst_hbm, hsa_hbm, hsb_hbm, zeros_hbm,
             out_hbm, sidx_v, didx_v, rows_v, zstage_v, acc_s):
  c = lax.axis_index("c")
  s = lax.axis_index("s")
  pltpu.sync_copy(zeros_hbm, zstage_v)
  r0 = s * ROWS_PER_TILE
  for k in range(ROWS_PER_TILE // ZROWS):
    pltpu.sync_copy(zstage_v, acc_s.at[pl.ds(r0 + k * ZROWS, ZROWS)])
  plsc.subcore_barrier()

  if col_split:
    base, n_my = _my_chunk_range(s, NCHUNKS, 0)
  else:
    base, n_my = _my_chunk_range(s, NCHUNKS // NC, c * (NCHUNKS // NC))

  def make_chunk_body(h_hbm):
    def chunk_body(i, carry):
      cb = base + i
      pltpu.sync_copy(src_hbm.at[cb], sidx_v)
      pltpu.sync_copy(dst_hbm.at[cb], didx_v)
      pltpu.sync_copy(h_hbm.at[sidx_v], rows_v)
      pltpu.sync_copy(rows_v, acc_s.at[didx_v], add=True)
      return carry
    return chunk_body

  @pl.when(c == 0)
  def _():
    lax.fori_loop(0, n_my, make_chunk_body(hsa_hbm), 0)

  @pl.when(c == 1)
  def _():
    lax.fori_loop(0, n_my, make_chunk_body(hsb_hbm), 0)

  plsc.subcore_barrier()

  for k in range(ROWS_PER_TILE // ZROWS):
    rr = r0 + k * ZROWS
    pltpu.sync_copy(acc_s.at[pl.ds(rr, ZROWS)], zstage_v)
    pltpu.sync_copy(zstage_v, out_hbm.at[c, pl.ds(rr, ZROWS)])


def _mp_call(col_split, src2d, dst2d, hs_a, hs_b, zeros_arr):
  return pl.kernel(
      functools.partial(_mp_body, col_split),
      out_type=jax.ShapeDtypeStruct((NC, N, D2), jnp.float32),
      mesh=_mesh(),
      scratch_types=[
          pltpu.VMEM((CHUNK,), jnp.int32),
          pltpu.VMEM((CHUNK,), jnp.int32),
          pltpu.VMEM((CHUNK, D2), jnp.float32),
          pltpu.VMEM((ZROWS, D2), jnp.float32),
          pltpu.VMEM_SHARED((N, D2), jnp.float32),
      ],
      name="sc_mp_colsplit" if col_split else "sc_mp_edgesplit",
  )(src2d, dst2d, hs_a, hs_b, zeros_arr)


def _dis_block(dp_ref):
  deg = dp_ref[0, :, 0:1] + dp_ref[1, :, 0:1] + 1.0
  return lax.rsqrt(deg)


def _tc1_body(x_ref, w1_ref, dp_ref, hsa_ref, hsb_ref):
  dis = _dis_block(dp_ref)
  h = lax.dot_general(x_ref[...], w1_ref[...], (((1,), (1,)), ((), ())),
                      preferred_element_type=jnp.float32)
  hs = h * dis
  hsa_ref[...] = hs[:, :D2]
  hsb_ref[...] = hs[:, D2:]


def _tc1_call(x, W1, degparts):
  return pl.pallas_call(
      _tc1_body,
      grid=(NBLK,),
      in_specs=[
          pl.BlockSpec((RBLK, 128), lambda i: (i, 0)),
          pl.BlockSpec((D1, 128), lambda i: (0, 0)),
          pl.BlockSpec((NC, RBLK, 16), lambda i: (0, i, 0)),
      ],
      out_specs=[
          pl.BlockSpec((RBLK, D2), lambda i: (i, 0)),
          pl.BlockSpec((RBLK, D2), lambda i: (i, 0)),
      ],
      out_shape=[
          jax.ShapeDtypeStruct((N, D2), jnp.float32),
          jax.ShapeDtypeStruct((N, D2), jnp.float32),
      ],
      name="tc1_matmul_scale",
  )(x, W1, degparts)


def _tc2_body(ms_ref, hsa_ref, hsb_ref, dp_ref, b1_ref, w2_ref, hs2_ref):
  dis = _dis_block(dp_ref)
  b1 = b1_ref[...]
  ha = jax.nn.relu(dis * (ms_ref[0] + hsa_ref[...]) + b1[:, :D2])
  hb = jax.nn.relu(dis * (ms_ref[1] + hsb_ref[...]) + b1[:, D2:])
  h1 = jnp.concatenate([ha, hb], axis=1)
  h2 = lax.dot_general(h1, w2_ref[...], (((1,), (1,)), ((), ())),
                       preferred_element_type=jnp.float32)
  hs2_ref[...] = h2 * dis


def _tc2_call(msgsum1, hs_a, hs_b, degparts, b1r, W2):
  return pl.pallas_call(
      _tc2_body,
      grid=(NBLK,),
      in_specs=[
          pl.BlockSpec((NC, RBLK, D2), lambda i: (0, i, 0)),
          pl.BlockSpec((RBLK, D2), lambda i: (i, 0)),
          pl.BlockSpec((RBLK, D2), lambda i: (i, 0)),
          pl.BlockSpec((NC, RBLK, 16), lambda i: (0, i, 0)),
          pl.BlockSpec((1, D1), lambda i: (0, 0)),
          pl.BlockSpec((D2, D1), lambda i: (0, 0)),
      ],
      out_specs=pl.BlockSpec((RBLK, D2), lambda i: (i, 0)),
      out_shape=jax.ShapeDtypeStruct((N, D2), jnp.float32),
      name="tc2_update_matmul",
  )(msgsum1, hs_a, hs_b, degparts, b1r, W2)


def _tc3_body(ms_ref, hs2_ref, dp_ref, b2_ref, batch_ref,
              wg_ref, bg_ref, wf_ref, bf_ref, wo_ref, bo_ref,
              o_ref, acc_ref):
  i = pl.program_id(0)

  @pl.when(i == 0)
  def _():
    acc_ref[...] = jnp.full((G, D2), -jnp.inf, dtype=jnp.float32)

  dis = _dis_block(dp_ref)
  h = jax.nn.relu(dis * (ms_ref[0] + ms_ref[1] + hs2_ref[...])
                  + b2_ref[...])
  b = batch_ref[0, 0, :]
  glo = jnp.min(b)
  ghi = jnp.max(b)

  def seg_body(g, carry):
    m = (b == g)[:, None]
    v = jnp.max(jnp.where(m, h, -jnp.inf), axis=0, keepdims=True)
    acc_ref[pl.ds(g, 1), :] = jnp.maximum(acc_ref[pl.ds(g, 1), :], v)
    return carry

  lax.fori_loop(glo, ghi + 1, seg_body, 0)

  @pl.when(i == NBLK - 1)
  def _():
    g0 = acc_ref[...]
    g1 = jax.nn.relu(
        lax.dot_general(g0, wg_ref[...], (((1,), (1,)), ((), ())),
                        preferred_element_type=jnp.float32) + bg_ref[...])
    g2 = jax.nn.relu(
        lax.dot_general(g1, wf_ref[...], (((1,), (1,)), ((), ())),
                        preferred_element_type=jnp.float32) + bf_ref[...])
    o_ref[...] = lax.dot_general(
        g2, wo_ref[...], (((1,), (1,)), ((), ())),
        preferred_element_type=jnp.float32) + bo_ref[...]


def _tc3_call(msgsum2, hs2, degparts, b2r, batch3, Wg, bgr, Wf, bfr, Wo, bor):
  return pl.pallas_call(
      _tc3_body,
      grid=(NBLK,),
      in_specs=[
          pl.BlockSpec((NC, RBLK, D2), lambda i: (0, i, 0)),
          pl.BlockSpec((RBLK, D2), lambda i: (i, 0)),
          pl.BlockSpec((NC, RBLK, 16), lambda i: (0, i, 0)),
          pl.BlockSpec((1, D2), lambda i: (0, 0)),
          pl.BlockSpec((1, 1, RBLK), lambda i: (i, 0, 0)),
          pl.BlockSpec((84, D2), lambda i: (0, 0)),
          pl.BlockSpec((1, 84), lambda i: (0, 0)),
          pl.BlockSpec((42, 84), lambda i: (0, 0)),
          pl.BlockSpec((1, 42), lambda i: (0, 0)),
          pl.BlockSpec((1, 42), lambda i: (0, 0)),
          pl.BlockSpec((1, 1), lambda i: (0, 0)),
      ],
      out_specs=pl.BlockSpec((G, 1), lambda i: (0, 0)),
      out_shape=jax.ShapeDtypeStruct((G, 1), jnp.float32),
      scratch_shapes=[pltpu.VMEM((G, D2), jnp.float32)],
      name="tc3_pool_mlp",
  )(msgsum2, hs2, degparts, b2r, batch3, Wg, bgr, Wf, bfr, Wo, bor)


def kernel(x, edge_index, batch, W1, b1, W2, b2, Wg, bg, Wf, bf, Wo, bo):
  src2d = edge_index[0].reshape(NCHUNKS, CHUNK)
  dst2d = edge_index[1].reshape(NCHUNKS, CHUNK)
  ones16 = jnp.ones((CHUNK, 16), jnp.float32)
  zeros16 = jnp.zeros((ZROWS, 16), jnp.float32)
  zeros168 = jnp.zeros((ZROWS, D2), jnp.float32)
  batch3 = batch.reshape(NBLK, 1, RBLK)
  b1r = b1.reshape(1, D1)
  b2r = b2.reshape(1, D2)
  bgr = bg.reshape(1, 84)
  bfr = bf.reshape(1, 42)
  bor = bo.reshape(1, 1)

  degparts = _deg_call(dst2d, ones16, zeros16)
  hs_a, hs_b = _tc1_call(x, W1, degparts)
  msgsum1 = _mp_call(True, src2d, dst2d, hs_a, hs_b, zeros168)
  hs2 = _tc2_call(msgsum1, hs_a, hs_b, degparts, b1r, W2)
  msgsum2 = _mp_call(False, src2d, dst2d, hs2, hs2, zeros168)
  return _tc3_call(msgsum2, hs2, degparts, b2r, batch3,
                   Wg, bgr, Wf, bfr, Wo, bor)


# speedup vs baseline: 8.6882x; 8.6882x over previous
import functools

import jax
import jax.numpy as jnp
from jax import lax
from jax.experimental import pallas as pl
from jax.experimental.pallas import tpu as pltpu
from jax.experimental.pallas import tpu_sc as plsc

N = 10000
E = 320000
G = 64
D1 = 336
D2 = 168
CW = 128
NCH1 = 3
NCH2 = 2
D1P = NCH1 * CW
D2P = NCH2 * CW
WD = 128
CHUNK = 128
NCHUNKS = E // CHUNK
NC = 2
NS = 16
GROUPS = N // 8
GPER, GREM = divmod(GROUPS, NS)
SGRP = 26
SROWS = SGRP * 8
RBLK = 1000
NBLK = N // RBLK

_mesh = functools.partial(
    plsc.VectorSubcoreMesh, core_axis_name="c", subcore_axis_name="s",
    num_cores=NC, num_subcores=NS)


def _my_chunk_range(sid, per_sc, sc_chunk0, nsplit=NS):
  per, rem = divmod(per_sc, nsplit)
  n_my = per + jnp.where(sid < rem, 1, 0)
  base = sc_chunk0 + sid * per + jnp.minimum(sid, rem)
  return base, n_my, per + (1 if rem else 0)


def _tile_rows(s):
  base_g = s * GPER + jnp.minimum(s, GREM)
  return base_g * 8


def _copy_rows_out(s, acc_s, stage_v, write_fn):
  r0 = _tile_rows(s)
  for k in range(GPER // SGRP):
    rr = pl.multiple_of(r0 + k * SROWS, 8)
    pltpu.sync_copy(acc_s.at[pl.ds(rr, SROWS)], stage_v)
    write_fn(stage_v, rr, SROWS)

  @pl.when(s < GREM)
  def _():
    rr = pl.multiple_of(r0 + GPER * 8, 8)
    pltpu.sync_copy(acc_s.at[pl.ds(rr, 8)], stage_v.at[pl.ds(0, 8)])
    write_fn(stage_v.at[pl.ds(0, 8)], rr, 8)


def _zero_rows(s, acc_s, zstage_v):
  r0 = _tile_rows(s)
  for k in range(GPER // SGRP):
    rr = pl.multiple_of(r0 + k * SROWS, 8)
    pltpu.sync_copy(zstage_v, acc_s.at[pl.ds(rr, SROWS)])

  @pl.when(s < GREM)
  def _():
    rr = pl.multiple_of(r0 + GPER * 8, 8)
    pltpu.sync_copy(zstage_v.at[pl.ds(0, 8)], acc_s.at[pl.ds(rr, 8)])


def _deg_body(dst_hbm, ones_hbm, zeros_hbm, out_hbm,
              idx_v, ones_v, zstage_v, acc_s):
  c = lax.axis_index("c")
  s = lax.axis_index("s")
  wid = c * NS + s
  pltpu.sync_copy(ones_hbm, ones_v)
  pltpu.sync_copy(zeros_hbm, zstage_v)
  _zero_rows(s, acc_s, zstage_v)
  plsc.subcore_barrier()

  base, n_my, nmax = _my_chunk_range(wid, NCHUNKS, 0, nsplit=NC * NS)

  def chunk_body(i, carry):
    @pl.when(i < n_my)
    def _():
      off = pl.multiple_of((base + i) * CHUNK, CHUNK)
      pltpu.sync_copy(dst_hbm.at[pl.ds(off, CHUNK)], idx_v)
      pltpu.sync_copy(ones_v, acc_s.at[idx_v], add=True)
    return carry

  lax.fori_loop(0, nmax, chunk_body, 0)
  plsc.subcore_barrier()

  def write_fn(stg, rr, nrows):
    pltpu.sync_copy(stg, out_hbm.at[c, pl.ds(rr, nrows)])

  _copy_rows_out(s, acc_s, zstage_v, write_fn)


def _deg_call(dst1d, ones_arr, zeros_arr):
  return pl.kernel(
      _deg_body,
      out_type=jax.ShapeDtypeStruct((NC, N, WD), jnp.float32),
      mesh=_mesh(),
      scratch_types=[
          pltpu.VMEM((CHUNK,), jnp.int32),
          pltpu.VMEM((CHUNK, WD), jnp.float32),
          pltpu.VMEM((SROWS, WD), jnp.float32),
          pltpu.VMEM_SHARED((N, WD), jnp.float32),
      ],
      name="sc_deg_hist",
  )(dst1d, ones_arr, zeros_arr)


def _mp_body(nsrc, phases, *refs):
  h_hbms = refs[:nsrc]
  src_hbm, dst_hbm, zeros_hbm, out_hbm = refs[nsrc:nsrc + 4]
  sidx_v, didx_v, rows_v, zstage_v, acc_s = refs[nsrc + 4:]
  c = lax.axis_index("c")
  s = lax.axis_index("s")
  pltpu.sync_copy(zeros_hbm, zstage_v)

  for (h_by_core, chunk0_by_core, per_sc, out_j_by_core) in phases:
    _zero_rows(s, acc_s, zstage_v)
    plsc.subcore_barrier()

    def make_loop(h_hbm, chunk0):
      base, n_my, nmax = _my_chunk_range(s, per_sc, chunk0)

      def chunk_body(i, carry):
        @pl.when(i < n_my)
        def _():
          off = pl.multiple_of((base + i) * CHUNK, CHUNK)
          pltpu.sync_copy(dst_hbm.at[pl.ds(off, CHUNK)], didx_v)
          pltpu.sync_copy(src_hbm.at[pl.ds(off, CHUNK)], sidx_v)
          pltpu.sync_copy(h_hbm.at[sidx_v], rows_v)
          pltpu.sync_copy(rows_v, acc_s.at[didx_v], add=True)
        return carry

      lax.fori_loop(0, nmax, chunk_body, 0)

    for cc in range(NC):
      @pl.when(c == cc)
      def _(cc=cc):
        make_loop(h_hbms[h_by_core[cc]], chunk0_by_core[cc])

    plsc.subcore_barrier()

    for cc in range(NC):
      @pl.when(c == cc)
      def _(cc=cc):
        def write_fn(stg, rr, nrows):
          pltpu.sync_copy(stg, out_hbm.at[out_j_by_core[cc], pl.ds(rr, nrows)])
        _copy_rows_out(s, acc_s, zstage_v, write_fn)


def _mp1_call(src1d, dst1d, hs_list, zeros_arr):
  half = NCHUNKS // NC
  outs = []
  for j in range(NCH1):
    phases = [((0, 0), (0, half), half, (0, 1))]
    outs.append(pl.kernel(
        functools.partial(_mp_body, 1, phases),
        out_type=jax.ShapeDtypeStruct((NC, N, CW), jnp.float32),
        mesh=_mesh(),
        scratch_types=[
            pltpu.VMEM((CHUNK,), jnp.int32),
            pltpu.VMEM((CHUNK,), jnp.int32),
            pltpu.VMEM((CHUNK, CW), jnp.float32),
            pltpu.VMEM((SROWS, CW), jnp.float32),
            pltpu.VMEM_SHARED((N, CW), jnp.float32),
        ],
        name=f"sc_mp1_c{j}",
    )(hs_list[j], src1d, dst1d, zeros_arr))
  return outs


def _mp2_call(src1d, dst1d, hs_list, zeros_arr):
  phases = [((0, 1), (0, 0), NCHUNKS, (0, 1))]
  return pl.kernel(
      functools.partial(_mp_body, NCH2, phases),
      out_type=jax.ShapeDtypeStruct((NCH2, N, CW), jnp.float32),
      mesh=_mesh(),
      scratch_types=[
          pltpu.VMEM((CHUNK,), jnp.int32),
          pltpu.VMEM((CHUNK,), jnp.int32),
          pltpu.VMEM((CHUNK, CW), jnp.float32),
          pltpu.VMEM((SROWS, CW), jnp.float32),
          pltpu.VMEM_SHARED((N, CW), jnp.float32),
      ],
      name="sc_mp2",
  )(*hs_list, src1d, dst1d, zeros_arr)


def _dis_block(dp_ref):
  deg = dp_ref[0, :, 0:1] + dp_ref[1, :, 0:1] + 1.0
  return lax.rsqrt(deg)


def _tc1_body(x_ref, w1_ref, dp_ref, hsa_ref, hsb_ref, hsc_ref):
  dis = _dis_block(dp_ref)
  h = lax.dot_general(x_ref[...], w1_ref[...], (((1,), (1,)), ((), ())),
                      preferred_element_type=jnp.float32)
  hs = h * dis
  hsa_ref[...] = hs[:, :CW]
  hsb_ref[...] = hs[:, CW:2 * CW]
  hsc_ref[...] = hs[:, 2 * CW:]


def _tc1_call(x, W1p, degparts):
  return pl.pallas_call(
      _tc1_body,
      grid=(NBLK,),
      in_specs=[
          pl.BlockSpec((RBLK, 128), lambda i: (i, 0)),
          pl.BlockSpec((D1P, 128), lambda i: (0, 0)),
          pl.BlockSpec((NC, RBLK, WD), lambda i: (0, i, 0)),
      ],
      out_specs=[
          pl.BlockSpec((RBLK, CW), lambda i: (i, 0)),
          pl.BlockSpec((RBLK, CW), lambda i: (i, 0)),
          pl.BlockSpec((RBLK, CW), lambda i: (i, 0)),
      ],
      out_shape=[
          jax.ShapeDtypeStruct((N, CW), jnp.float32),
          jax.ShapeDtypeStruct((N, CW), jnp.float32),
          jax.ShapeDtypeStruct((N, CW), jnp.float32),
      ],
      name="tc1_matmul_scale",
  )(x, W1p, degparts)


def _tc2_body(ms0_ref, ms1_ref, ms2_ref, hsa_ref, hsb_ref, hsc_ref,
              dp_ref, b1_ref, w2_ref, hs2a_ref, hs2b_ref):
  dis = _dis_block(dp_ref)
  msf = jnp.concatenate(
      [ms0_ref[0] + ms0_ref[1], ms1_ref[0] + ms1_ref[1],
       ms2_ref[0] + ms2_ref[1]], axis=1)[:, :D1]
  hsf = jnp.concatenate([hsa_ref[...], hsb_ref[...], hsc_ref[...]],
                        axis=1)[:, :D1]
  h1 = jax.nn.relu(dis * (msf + hsf) + b1_ref[...])
  h2 = lax.dot_general(h1, w2_ref[...], (((1,), (1,)), ((), ())),
                       preferred_element_type=jnp.float32)
  hs2 = h2 * dis
  hs2a_ref[...] = hs2[:, :CW]
  hs2b_ref[...] = jnp.concatenate(
      [hs2[:, CW:], jnp.zeros((RBLK, D2P - D2), jnp.float32)], axis=1)


def _tc2_call(msgsum1, hs_list, degparts, b1r, W2):
  return pl.pallas_call(
      _tc2_body,
      grid=(NBLK,),
      in_specs=[
          pl.BlockSpec((NC, RBLK, CW), lambda i: (0, i, 0)),
          pl.BlockSpec((NC, RBLK, CW), lambda i: (0, i, 0)),
          pl.BlockSpec((NC, RBLK, CW), lambda i: (0, i, 0)),
          pl.BlockSpec((RBLK, CW), lambda i: (i, 0)),
          pl.BlockSpec((RBLK, CW), lambda i: (i, 0)),
          pl.BlockSpec((RBLK, CW), lambda i: (i, 0)),
          pl.BlockSpec((NC, RBLK, WD), lambda i: (0, i, 0)),
          pl.BlockSpec((1, D1), lambda i: (0, 0)),
          pl.BlockSpec((D2, D1), lambda i: (0, 0)),
      ],
      out_specs=[
          pl.BlockSpec((RBLK, CW), lambda i: (i, 0)),
          pl.BlockSpec((RBLK, CW), lambda i: (i, 0)),
      ],
      out_shape=[
          jax.ShapeDtypeStruct((N, CW), jnp.float32),
          jax.ShapeDtypeStruct((N, CW), jnp.float32),
      ],
      name="tc2_update_matmul",
  )(*msgsum1, *hs_list, degparts, b1r, W2)


def _tc3_body(ms_ref, hs2a_ref, hs2b_ref, dp_ref, b2_ref, batch_ref,
              wg_ref, bg_ref, wf_ref, bf_ref, wo_ref, bo_ref,
              o_ref, acc_ref):
  i = pl.program_id(0)

  @pl.when(i == 0)
  def _():
    acc_ref[...] = jnp.full((G, D2), -jnp.inf, dtype=jnp.float32)

  dis = _dis_block(dp_ref)
  msf = jnp.concatenate([ms_ref[0], ms_ref[1]], axis=1)[:, :D2]
  hsf = jnp.concatenate([hs2a_ref[...], hs2b_ref[...]], axis=1)[:, :D2]
  h = jax.nn.relu(dis * (msf + hsf) + b2_ref[...])
  b = batch_ref[...]
  glo = jnp.min(b)
  ghi = jnp.max(b)

  def seg_body(g, carry):
    m = (b == g)
    v = jnp.max(jnp.where(m, h, -jnp.inf), axis=0, keepdims=True)
    acc_ref[pl.ds(g, 1), :] = jnp.maximum(acc_ref[pl.ds(g, 1), :], v)
    return carry

  lax.fori_loop(glo, ghi + 1, seg_body, 0)

  @pl.when(i == NBLK - 1)
  def _():
    g0 = acc_ref[...]
    g1 = jax.nn.relu(
        lax.dot_general(g0, wg_ref[...], (((1,), (1,)), ((), ())),
                        preferred_element_type=jnp.float32) + bg_ref[...])
    g2 = jax.nn.relu(
        lax.dot_general(g1, wf_ref[...], (((1,), (1,)), ((), ())),
                        preferred_element_type=jnp.float32) + bf_ref[...])
    res = lax.dot_general(
        g2, wo_ref[...], (((1,), (1,)), ((), ())),
        preferred_element_type=jnp.float32)
    o_ref[...] = res[:, 0:1] + bo_ref[0, 0]


def _tc3_call(msgsum2, hs2_list, degparts, b2r, batch2,
              Wg, bgr, Wf, bfr, Wo, bor):
  return pl.pallas_call(
      _tc3_body,
      grid=(NBLK,),
      in_specs=[
          pl.BlockSpec((NCH2, RBLK, CW), lambda i: (0, i, 0)),
          pl.BlockSpec((RBLK, CW), lambda i: (i, 0)),
          pl.BlockSpec((RBLK, CW), lambda i: (i, 0)),
          pl.BlockSpec((NC, RBLK, WD), lambda i: (0, i, 0)),
          pl.BlockSpec((1, D2), lambda i: (0, 0)),
          pl.BlockSpec((RBLK, 1), lambda i: (i, 0)),
          pl.BlockSpec((84, D2), lambda i: (0, 0)),
          pl.BlockSpec((1, 84), lambda i: (0, 0)),
          pl.BlockSpec((42, 84), lambda i: (0, 0)),
          pl.BlockSpec((1, 42), lambda i: (0, 0)),
          pl.BlockSpec((8, 42), lambda i: (0, 0)),
          pl.BlockSpec((1, 1), lambda i: (0, 0)),
      ],
      out_specs=pl.BlockSpec((G, 1), lambda i: (0, 0)),
      out_shape=jax.ShapeDtypeStruct((G, 1), jnp.float32),
      scratch_shapes=[pltpu.VMEM((G, D2), jnp.float32)],
      name="tc3_pool_mlp",
  )(msgsum2, *hs2_list, degparts, b2r, batch2,
    Wg, bgr, Wf, bfr, Wo, bor)


def kernel(x, edge_index, batch, W1, b1, W2, b2, Wg, bg, Wf, bf, Wo, bo):
  src1d = edge_index[0]
  dst1d = edge_index[1]
  W1p = jnp.pad(W1, ((0, D1P - D1), (0, 0)))
  oneswd = jnp.ones((CHUNK, WD), jnp.float32)
  zeroswd = jnp.zeros((SROWS, WD), jnp.float32)
  zeroscw = jnp.zeros((SROWS, CW), jnp.float32)
  batch2 = batch.reshape(N, 1)
  b1r = b1.reshape(1, D1)
  b2r = b2.reshape(1, D2)
  bgr = bg.reshape(1, 84)
  bfr = bf.reshape(1, 42)
  bor = bo.reshape(1, 1)

  Wop = jnp.pad(Wo, ((0, 7), (0, 0)))
  degparts = _deg_call(dst1d, oneswd, zeroswd)
  hs_list = _tc1_call(x, W1p, degparts)
  msgsum1 = _mp1_call(src1d, dst1d, hs_list, zeroscw)
  hs2_list = _tc2_call(msgsum1, hs_list, degparts, b1r, W2)
  msgsum2 = _mp2_call(src1d, dst1d, hs2_list, zeroscw)
  return _tc3_call(msgsum2, hs2_list, degparts, b2r, batch2,
                   Wg, bgr, Wf, bfr, Wop, bor)
